# baseline (speedup 1.0000x reference)
import jax
import jax.numpy as jnp
from jax.experimental import pallas as pl
from jax.experimental.pallas import tpu as pltpu

EMB = 1024
TB = 512


def _mlp_block(x_ref, w1_ref, b1_ref, w2_ref, out_ref, w2s_ref):
    @pl.when(pl.program_id(0) == 0)
    def _():
        w2s_ref[...] = (0.5 * w2_ref[...]).astype(jnp.bfloat16)

    x = x_ref[...]
    x = jnp.where(jnp.abs(x) < 1e-06, 0.0, x) * 0.5
    t = (jnp.dot(x, w1_ref[...], preferred_element_type=jnp.float32)
         + 0.5 * b1_ref[...])
    s = t + t * jnp.tanh(t)
    y = jnp.dot(s.astype(jnp.bfloat16), w2s_ref[...],
                preferred_element_type=jnp.float32)
    out_ref[...] = 8.0 * jnp.tanh(y) + 8.0


def kernel(pos1, pos2, W1, b1, W2):
    B, S = pos1.shape
    n = B * S
    x = jnp.stack((pos1.reshape(n), pos2.reshape(n)), axis=-1)
    grid = n // TB
    out = pl.pallas_call(
        _mlp_block,
        grid=(grid,),
        in_specs=[
            pl.BlockSpec((TB, 2), lambda i: (i, 0)),
            pl.BlockSpec((2, EMB), lambda i: (0, 0)),
            pl.BlockSpec((1, EMB), lambda i: (0, 0)),
            pl.BlockSpec((EMB, EMB), lambda i: (0, 0)),
        ],
        out_specs=pl.BlockSpec((TB, EMB), lambda i: (i, 0)),
        out_shape=jax.ShapeDtypeStruct((n, EMB), jnp.float32),
        scratch_shapes=[pltpu.VMEM((EMB, EMB), jnp.bfloat16)],
        compiler_params=pltpu.CompilerParams(
            dimension_semantics=("arbitrary",),
        ),
    )(x, W1, b1.reshape(1, EMB), W2)
    return out.reshape(B, S, EMB)


# speedup vs baseline: 1.0107x; 1.0107x over previous
import jax
import jax.numpy as jnp
from jax.experimental import pallas as pl
from jax.experimental.pallas import tpu as pltpu

EMB = 1024
TB = 2048


def _mlp_block(x_ref, w1_ref, b1_ref, w2_ref, out_ref, w2s_ref):
    @pl.when(pl.program_id(0) == 0)
    def _():
        w2s_ref[...] = (0.5 * w2_ref[...]).astype(jnp.bfloat16)

    x = x_ref[...]
    x = jnp.where(jnp.abs(x) < 1e-06, 0.0, x) * 0.5
    t = (jnp.dot(x, w1_ref[...], preferred_element_type=jnp.float32)
         + 0.5 * b1_ref[...])
    s = t + t * jnp.tanh(t)
    y = jnp.dot(s.astype(jnp.bfloat16), w2s_ref[...],
                preferred_element_type=jnp.float32)
    out_ref[...] = 8.0 * jnp.tanh(y) + 8.0


def kernel(pos1, pos2, W1, b1, W2):
    B, S = pos1.shape
    n = B * S
    x = jnp.stack((pos1.reshape(n), pos2.reshape(n)), axis=-1)
    grid = n // TB
    out = pl.pallas_call(
        _mlp_block,
        grid=(grid,),
        in_specs=[
            pl.BlockSpec((TB, 2), lambda i: (i, 0)),
            pl.BlockSpec((2, EMB), lambda i: (0, 0)),
            pl.BlockSpec((1, EMB), lambda i: (0, 0)),
            pl.BlockSpec((EMB, EMB), lambda i: (0, 0)),
        ],
        out_specs=pl.BlockSpec((TB, EMB), lambda i: (i, 0)),
        out_shape=jax.ShapeDtypeStruct((n, EMB), jnp.float32),
        scratch_shapes=[pltpu.VMEM((EMB, EMB), jnp.bfloat16)],
        compiler_params=pltpu.CompilerParams(
            dimension_semantics=("arbitrary",),
        ),
    )(x, W1, b1.reshape(1, EMB), W2)
    return out.reshape(B, S, EMB)


# speedup vs baseline: 1.0126x; 1.0019x over previous
import jax
import jax.numpy as jnp
from jax.experimental import pallas as pl
from jax.experimental.pallas import tpu as pltpu

EMB = 1024
TB = 1024


def _mlp_block(p1_ref, p2_ref, w1_ref, b1_ref, w2_ref, out_ref, w2s_ref):
    @pl.when(pl.program_id(0) == 0)
    def _():
        w2s_ref[...] = (0.5 * w2_ref[...]).astype(jnp.bfloat16)

    p1 = p1_ref[...]
    p2 = p2_ref[...]
    p1 = jnp.where(jnp.abs(p1) < 1e-06, 0.0, p1) * 0.5
    p2 = jnp.where(jnp.abs(p2) < 1e-06, 0.0, p2) * 0.5
    t = p1 * w1_ref[0:1, :] + (p2 * w1_ref[1:2, :] + 0.5 * b1_ref[...])
    s = t + t * jnp.tanh(t)
    y = jnp.dot(s.astype(jnp.bfloat16), w2s_ref[...],
                preferred_element_type=jnp.float32)
    out_ref[...] = 8.0 * jnp.tanh(y) + 8.0


def kernel(pos1, pos2, W1, b1, W2):
    B, S = pos1.shape
    n = B * S
    grid = n // TB
    out = pl.pallas_call(
        _mlp_block,
        grid=(grid,),
        in_specs=[
            pl.BlockSpec((TB, 1), lambda i: (i, 0)),
            pl.BlockSpec((TB, 1), lambda i: (i, 0)),
            pl.BlockSpec((2, EMB), lambda i: (0, 0)),
            pl.BlockSpec((1, EMB), lambda i: (0, 0)),
            pl.BlockSpec((EMB, EMB), lambda i: (0, 0)),
        ],
        out_specs=pl.BlockSpec((TB, EMB), lambda i: (i, 0)),
        out_shape=jax.ShapeDtypeStruct((n, EMB), jnp.float32),
        scratch_shapes=[pltpu.VMEM((EMB, EMB), jnp.bfloat16)],
        compiler_params=pltpu.CompilerParams(
            dimension_semantics=("arbitrary",),
        ),
    )(pos1.reshape(n, 1), pos2.reshape(n, 1), W1, b1.reshape(1, EMB), W2)
    return out.reshape(B, S, EMB)


# speedup vs baseline: 1.0288x; 1.0160x over previous
import jax
import jax.numpy as jnp
from jax.experimental import pallas as pl
from jax.experimental.pallas import tpu as pltpu

EMB = 1024
TB = 1024


def _mlp_block(x_ref, w1_ref, b1_ref, w2_ref, out_ref, w2s_ref):
    @pl.when(pl.program_id(0) == 0)
    def _():
        w2s_ref[...] = (0.5 * w2_ref[...]).astype(jnp.bfloat16)

    x = x_ref[...]
    x = jnp.where(jnp.abs(x) < 1e-06, 0.0, x) * 0.5
    t = (jnp.dot(x, w1_ref[...], preferred_element_type=jnp.float32)
         + 0.5 * b1_ref[...])

    H = TB // 2
    tA, tB = t[:H], t[H:]
    sA = (tA + tA * jnp.tanh(tA)).astype(jnp.bfloat16)
    yA = jnp.dot(sA, w2s_ref[...], preferred_element_type=jnp.float32)
    sB = (tB + tB * jnp.tanh(tB)).astype(jnp.bfloat16)
    out_ref[:H, :] = 8.0 * jnp.tanh(yA) + 8.0
    yB = jnp.dot(sB, w2s_ref[...], preferred_element_type=jnp.float32)
    out_ref[H:, :] = 8.0 * jnp.tanh(yB) + 8.0


def kernel(pos1, pos2, W1, b1, W2):
    B, S = pos1.shape
    n = B * S
    x = jnp.stack((pos1.reshape(n), pos2.reshape(n)), axis=-1)
    grid = n // TB
    out = pl.pallas_call(
        _mlp_block,
        grid=(grid,),
        in_specs=[
            pl.BlockSpec((TB, 2), lambda i: (i, 0)),
            pl.BlockSpec((2, EMB), lambda i: (0, 0)),
            pl.BlockSpec((1, EMB), lambda i: (0, 0)),
            pl.BlockSpec((EMB, EMB), lambda i: (0, 0)),
        ],
        out_specs=pl.BlockSpec((TB, EMB), lambda i: (i, 0)),
        out_shape=jax.ShapeDtypeStruct((n, EMB), jnp.float32),
        scratch_shapes=[pltpu.VMEM((EMB, EMB), jnp.bfloat16)],
        compiler_params=pltpu.CompilerParams(
            dimension_semantics=("arbitrary",),
        ),
    )(x, W1, b1.reshape(1, EMB), W2)
    return out.reshape(B, S, EMB)


# speedup vs baseline: 1.0352x; 1.0062x over previous
import jax
import jax.numpy as jnp
from jax.experimental import pallas as pl
from jax.experimental.pallas import tpu as pltpu

EMB = 1024
TB = 1024


def _mlp_block(x_ref, w1_ref, b1_ref, w2_ref, out_ref, w2s_ref):
    @pl.when(pl.program_id(0) == 0)
    def _():
        w2s_ref[...] = (0.5 * w2_ref[...]).astype(jnp.bfloat16)

    x = x_ref[...]
    x = jnp.where(jnp.abs(x) < 1e-06, 0.0, x) * 0.5
    t = (jnp.dot(x.astype(jnp.bfloat16), w1_ref[...].astype(jnp.bfloat16),
                 preferred_element_type=jnp.float32)
         + 0.5 * b1_ref[...])
    s = t + t * jnp.tanh(t)
    y = jnp.dot(s.astype(jnp.bfloat16), w2s_ref[...],
                preferred_element_type=jnp.float32)
    out_ref[...] = 8.0 * jnp.tanh(y) + 8.0


def kernel(pos1, pos2, W1, b1, W2):
    B, S = pos1.shape
    n = B * S
    x = jnp.stack((pos1.reshape(n), pos2.reshape(n)), axis=-1)
    grid = n // TB
    out = pl.pallas_call(
        _mlp_block,
        grid=(grid,),
        in_specs=[
            pl.BlockSpec((TB, 2), lambda i: (i, 0)),
            pl.BlockSpec((2, EMB), lambda i: (0, 0)),
            pl.BlockSpec((1, EMB), lambda i: (0, 0)),
            pl.BlockSpec((EMB, EMB), lambda i: (0, 0)),
        ],
        out_specs=pl.BlockSpec((TB, EMB), lambda i: (i, 0)),
        out_shape=jax.ShapeDtypeStruct((n, EMB), jnp.float32),
        scratch_shapes=[pltpu.VMEM((EMB, EMB), jnp.bfloat16)],
        compiler_params=pltpu.CompilerParams(
            dimension_semantics=("arbitrary",),
        ),
    )(x, W1, b1.reshape(1, EMB), W2)
    return out.reshape(B, S, EMB)


# speedup vs baseline: 1.0645x; 1.0284x over previous
import jax
import jax.numpy as jnp
from jax.experimental import pallas as pl
from jax.experimental.pallas import tpu as pltpu

EMB = 1024
TB = 1024


def _mlp_block(x_ref, w1_ref, b1_ref, w2_ref, out_ref, w1s_ref, b1s_ref,
               w2s_ref):
    @pl.when(pl.program_id(0) == 0)
    def _():
        w1s_ref[...] = (0.5 * w1_ref[...]).astype(jnp.bfloat16)
        b1s_ref[...] = 0.5 * b1_ref[...]
        w2s_ref[...] = (0.5 * w2_ref[...]).astype(jnp.bfloat16)

    x = x_ref[...]
    x = jnp.where(jnp.abs(x) < 1e-06, 0.0, x)
    t = (jnp.dot(x, w1s_ref[...], preferred_element_type=jnp.float32)
         + b1s_ref[...])
    s = t + t * jnp.tanh(t)
    y = jnp.dot(s.astype(jnp.bfloat16), w2s_ref[...],
                preferred_element_type=jnp.float32)
    out_ref[...] = 8.0 * jnp.tanh(y) + 8.0


def kernel(pos1, pos2, W1, b1, W2):
    B, S = pos1.shape
    n = B * S
    x = jnp.stack((pos1.reshape(n), pos2.reshape(n)),
                  axis=-1).astype(jnp.bfloat16)
    grid = n // TB
    out = pl.pallas_call(
        _mlp_block,
        grid=(grid,),
        in_specs=[
            pl.BlockSpec((TB, 2), lambda i: (i, 0)),
            pl.BlockSpec((2, EMB), lambda i: (0, 0)),
            pl.BlockSpec((1, EMB), lambda i: (0, 0)),
            pl.BlockSpec((EMB, EMB), lambda i: (0, 0)),
        ],
        out_specs=pl.BlockSpec((TB, EMB), lambda i: (i, 0)),
        out_shape=jax.ShapeDtypeStruct((n, EMB), jnp.float32),
        scratch_shapes=[pltpu.VMEM((2, EMB), jnp.bfloat16),
                        pltpu.VMEM((1, EMB), jnp.float32),
                        pltpu.VMEM((EMB, EMB), jnp.bfloat16)],
        compiler_params=pltpu.CompilerParams(
            dimension_semantics=("arbitrary",),
        ),
    )(x, W1, b1.reshape(1, EMB), W2)
    return out.reshape(B, S, EMB)
